# baseline (speedup 1.0000x reference)
import functools

import jax
import jax.numpy as jnp
from jax import lax
from jax.experimental import pallas as pl
from jax.experimental.pallas import tpu as pltpu
from jax.experimental.pallas import tpu_sc as plsc

_ROWS = 64
_DIM = 64
_STRIDE = 15625
_LANES = 128
_ROWS_PER_TILE = 2
_NUM_ACTIVE = _ROWS // _ROWS_PER_TILE
_VEC = 16

_mesh = plsc.VectorSubcoreMesh(core_axis_name="c", subcore_axis_name="s")


@functools.partial(
    pl.kernel,
    out_type=jax.ShapeDtypeStruct((_ROWS * _DIM,), jnp.float32),
    mesh=_mesh,
    scratch_types=[
        pltpu.VMEM((_ROWS_PER_TILE, _DIM, _LANES), jnp.float32),
        pltpu.VMEM((_ROWS_PER_TILE * _DIM,), jnp.float32),
        pltpu.SemaphoreType.DMA,
    ],
    compiler_params=pltpu.CompilerParams(
        needs_layout_passes=False, skip_device_barrier=True
    ),
)
def _gather_cols(tt_hbm, out_hbm, blk_v, slab_v, sem):
    cid = lax.axis_index("c")
    sid = lax.axis_index("s")
    wid = sid * _mesh.num_cores + cid
    base = wid * _ROWS_PER_TILE

    def block_start(k):
        col = (base + k) * _STRIDE
        start = pl.multiple_of((col // _LANES) * _LANES, _LANES)
        return col, start

    for k in range(_ROWS_PER_TILE):
        _, start = block_start(k)
        pltpu.async_copy(tt_hbm.at[:, pl.ds(start, _LANES)], blk_v.at[k], sem)
    for k in range(_ROWS_PER_TILE):
        _, start = block_start(k)
        pltpu.make_async_copy(
            tt_hbm.at[:, pl.ds(start, _LANES)], blk_v.at[k], sem
        ).wait()
    for k in range(_ROWS_PER_TILE):
        col, start = block_start(k)
        lane_vec = jnp.full((_VEC,), col - start, dtype=jnp.int32)
        for g in range(_DIM // _VEC):
            feat = lax.iota(jnp.int32, _VEC) + g * _VEC
            vals = plsc.load_gather(blk_v.at[k], [feat, lane_vec])
            slab_v[pl.ds(k * _DIM + g * _VEC, _VEC)] = vals
    pltpu.sync_copy(
        slab_v, out_hbm.at[pl.ds(base * _DIM, _ROWS_PER_TILE * _DIM)]
    )


def kernel(table):
    return _gather_cols(table.T).reshape(_ROWS, _DIM)


# speedup vs baseline: 1.0067x; 1.0067x over previous
import functools

import jax
import jax.numpy as jnp
from jax import lax
from jax.experimental import pallas as pl
from jax.experimental.pallas import tpu as pltpu
from jax.experimental.pallas import tpu_sc as plsc

_ROWS = 64
_DIM = 64
_STRIDE = 15625
_LANES = 128
_ROWS_PER_TILE = 2
_NUM_ACTIVE = _ROWS // _ROWS_PER_TILE
_VEC = 16

_mesh = plsc.VectorSubcoreMesh(core_axis_name="c", subcore_axis_name="s")


@functools.partial(
    pl.kernel,
    out_type=jax.ShapeDtypeStruct((_ROWS * _DIM,), jnp.float32),
    mesh=_mesh,
    scratch_types=[
        pltpu.VMEM((_ROWS_PER_TILE, _DIM, _LANES), jnp.float32),
        pltpu.VMEM((_ROWS_PER_TILE * _DIM,), jnp.float32),
        pltpu.SemaphoreType.DMA,
    ],
    compiler_params=pltpu.CompilerParams(needs_layout_passes=False),
)
def _gather_cols(tt_hbm, out_hbm, blk_v, slab_v, sem):
    cid = lax.axis_index("c")
    sid = lax.axis_index("s")
    wid = sid * _mesh.num_cores + cid
    base = wid * _ROWS_PER_TILE

    def block_start(k):
        col = (base + k) * _STRIDE
        start = pl.multiple_of((col // _LANES) * _LANES, _LANES)
        return col, start

    for k in range(_ROWS_PER_TILE):
        _, start = block_start(k)
        pltpu.async_copy(tt_hbm.at[:, pl.ds(start, _LANES)], blk_v.at[k], sem)
    for k in range(_ROWS_PER_TILE):
        _, start = block_start(k)
        pltpu.make_async_copy(
            tt_hbm.at[:, pl.ds(start, _LANES)], blk_v.at[k], sem
        ).wait()
    for k in range(_ROWS_PER_TILE):
        col, start = block_start(k)
        lane_vec = jnp.full((_VEC,), col - start, dtype=jnp.int32)
        for g in range(_DIM // _VEC):
            feat = lax.iota(jnp.int32, _VEC) + g * _VEC
            vals = plsc.load_gather(blk_v.at[k], [feat, lane_vec])
            slab_v[pl.ds(k * _DIM + g * _VEC, _VEC)] = vals
    pltpu.sync_copy(
        slab_v, out_hbm.at[pl.ds(base * _DIM, _ROWS_PER_TILE * _DIM)]
    )


def kernel(table):
    return _gather_cols(table.T).reshape(_ROWS, _DIM)
